# baseline (speedup 1.0000x reference)
import jax
import jax.numpy as jnp
from jax.experimental import pallas as pl
from jax.experimental.pallas import tpu as pltpu
from jax.experimental.pallas import tpu_sc as plsc

_WINDOW = 512
_PCHUNK = 1024
_OCHUNK = 512


def _dotT(x, eye):
    return jax.lax.dot_general(
        x,
        eye,
        (((0,), (0,)), ((), ())),
        precision=jax.lax.Precision.HIGHEST,
        preferred_element_type=jnp.float32,
    )


def _pack_body(xa_ref, xb_ref, o_ref):
    eye = jnp.eye(xa_ref.shape[0], dtype=jnp.float32)
    o_ref[...] = jnp.concatenate(
        [_dotT(xa_ref[...], eye), _dotT(xb_ref[...], eye)], axis=1
    )


def _pack_table(wt, npairs):
    h, v = wt.shape
    maxb = -(-v // _PCHUNK) - 1
    return pl.pallas_call(
        _pack_body,
        grid=(npairs,),
        in_specs=[
            pl.BlockSpec((h, _PCHUNK), lambda i: (0, 2 * i)),
            pl.BlockSpec(
                (h, _PCHUNK), lambda i: (0, jnp.minimum(2 * i + 1, maxb))
            ),
        ],
        out_specs=pl.BlockSpec((_PCHUNK, 2 * h), lambda i: (i, 0)),
        out_shape=jax.ShapeDtypeStruct((npairs * _PCHUNK, 2 * h), wt.dtype),
    )(wt, wt)


def _make_untranspose_body(h):
    def body(x_ref, o_ref):
        x = x_ref[0]
        eye = jnp.eye(h, dtype=jnp.float32)
        ce = jax.lax.dot_general(
            eye, x[:, :h], (((1,), (1,)), ((), ())),
            precision=jax.lax.Precision.HIGHEST,
            preferred_element_type=jnp.float32,
        )
        co = jax.lax.dot_general(
            eye, x[:, h:], (((1,), (1,)), ((), ())),
            precision=jax.lax.Precision.HIGHEST,
            preferred_element_type=jnp.float32,
        )
        o_ref[0] = jnp.stack([ce, co], axis=2).reshape(h, 2 * x.shape[0])

    return body


def _rows_to_out(rows3, ll, b, h):
    hb = b // 2
    mblocks = hb // _OCHUNK
    return pl.pallas_call(
        _make_untranspose_body(h),
        grid=(ll, mblocks),
        in_specs=[
            pl.BlockSpec((1, _OCHUNK, 2 * h), lambda l, m: (l, m, 0))
        ],
        out_specs=pl.BlockSpec(
            (1, h, 2 * _OCHUNK), lambda l, m: (l, 0, m)
        ),
        out_shape=jax.ShapeDtypeStruct((ll, h, b), rows3.dtype),
    )(rows3)


def _sc_gather(table, idx, n, h):
    mesh = plsc.VectorSubcoreMesh(
        core_axis_name="core", subcore_axis_name="subcore"
    )

    @pl.kernel(
        out_type=jax.ShapeDtypeStruct((n, h), table.dtype),
        mesh=mesh,
        compiler_params=pltpu.CompilerParams(use_tc_tiling_on_sc=False),
    )
    def run(table_hbm, idx_hbm, out_hbm):
        def body(i_vmem, o_vmem):
            pltpu.sync_copy(table_hbm.at[i_vmem.at[0]], o_vmem)

        pltpu.emit_pipeline(
            body,
            grid=(n // _WINDOW,),
            in_specs=[
                pl.BlockSpec((1, _WINDOW), index_map=lambda i: (0, i))
            ],
            out_specs=[
                pl.BlockSpec((_WINDOW, h), index_map=lambda i: (i, 0))
            ],
            core_axis_name=("core", "subcore"),
            dimension_semantics=(pltpu.PARALLEL,),
        )(idx_hbm, out_hbm)

    return run(table, idx)


def kernel(inputs, weight):
    b, ll = inputs.shape
    v, h = weight.shape
    n = b * ll
    npairs = -(-v // (2 * _PCHUNK))
    vpad = npairs * 2 * _PCHUNK

    wt = weight.T

    packed = _pack_table(wt, npairs)
    table_lin = packed.reshape(vpad, h)

    e = inputs.T.astype(jnp.int32).reshape(1, n)
    be = e // _PCHUNK
    j = ((be // 2) * _PCHUNK + (e % _PCHUNK)) * 2 + (be % 2)

    rows = _sc_gather(table_lin, j, n, h)
    out_t = _rows_to_out(rows.reshape(ll, b // 2, 2 * h), ll, b, h)
    return jnp.transpose(out_t, (2, 0, 1))


# speedup vs baseline: 12.7613x; 12.7613x over previous
import jax
import jax.numpy as jnp
from jax.experimental import pallas as pl
from jax.experimental.pallas import tpu as pltpu
from jax.experimental.pallas import tpu_sc as plsc

_WINDOW = 512
_PCHUNK = 1024
_OCHUNK = 512


def _dotT(x, eye):
    return jax.lax.dot_general(
        x,
        eye,
        (((0,), (0,)), ((), ())),
        precision=jax.lax.Precision.HIGHEST,
        preferred_element_type=jnp.float32,
    )


def _pack_body(xa_ref, xb_ref, o_ref):
    o_ref[...] = jnp.concatenate([xa_ref[...].T, xb_ref[...].T], axis=1)


def _pack_table(wt, npairs):
    h, v = wt.shape
    maxb = -(-v // _PCHUNK) - 1
    return pl.pallas_call(
        _pack_body,
        grid=(npairs,),
        in_specs=[
            pl.BlockSpec((h, _PCHUNK), lambda i: (0, 2 * i)),
            pl.BlockSpec(
                (h, _PCHUNK), lambda i: (0, jnp.minimum(2 * i + 1, maxb))
            ),
        ],
        out_specs=pl.BlockSpec((_PCHUNK, 2 * h), lambda i: (i, 0)),
        out_shape=jax.ShapeDtypeStruct((npairs * _PCHUNK, 2 * h), wt.dtype),
    )(wt, wt)


def _make_untranspose_body(h, half_b):
    def body(x_ref, o_ref):
        x = x_ref[...]
        o_ref[0, :, :half_b] = x[:, :h].T
        o_ref[0, :, half_b:] = x[:, h:].T

    return body


def _rows_to_out(rows2, ll, b, h):
    hb = b // 2
    return pl.pallas_call(
        _make_untranspose_body(h, hb),
        grid=(ll,),
        in_specs=[pl.BlockSpec((hb, 2 * h), lambda l: (l, 0))],
        out_specs=pl.BlockSpec((1, h, b), lambda l: (l, 0, 0)),
        out_shape=jax.ShapeDtypeStruct((ll, h, b), rows2.dtype),
    )(rows2)


def _sc_gather(table, idx, n, h):
    mesh = plsc.VectorSubcoreMesh(
        core_axis_name="core", subcore_axis_name="subcore"
    )

    @pl.kernel(
        out_type=jax.ShapeDtypeStruct((n, h), table.dtype),
        mesh=mesh,
        compiler_params=pltpu.CompilerParams(use_tc_tiling_on_sc=False),
    )
    def run(table_hbm, idx_hbm, out_hbm):
        def body(i_vmem, o_vmem):
            pltpu.sync_copy(table_hbm.at[i_vmem.at[0]], o_vmem)

        pltpu.emit_pipeline(
            body,
            grid=(n // _WINDOW,),
            in_specs=[
                pl.BlockSpec((1, _WINDOW), index_map=lambda i: (0, i))
            ],
            out_specs=[
                pl.BlockSpec((_WINDOW, h), index_map=lambda i: (i, 0))
            ],
            core_axis_name=("core", "subcore"),
            dimension_semantics=(pltpu.PARALLEL,),
        )(idx_hbm, out_hbm)

    return run(table, idx)


def kernel(inputs, weight):
    b, ll = inputs.shape
    v, h = weight.shape
    n = b * ll
    npairs = -(-v // (2 * _PCHUNK))
    vpad = npairs * 2 * _PCHUNK

    wt = weight.T

    packed = _pack_table(wt, npairs)
    table_lin = packed.reshape(vpad, h)

    s = jnp.arange(b, dtype=jnp.int32)
    perm = (s % 2) * (b // 2) + s // 2
    idx_t = inputs.T.astype(jnp.int32)
    e = jnp.take(idx_t, perm, axis=1).reshape(1, n)
    be = e // _PCHUNK
    j = ((be // 2) * _PCHUNK + (e % _PCHUNK)) * 2 + (be % 2)

    rows = _sc_gather(table_lin, j, n, h)
    out_t = _rows_to_out(rows.reshape(n // 2, 2 * h), ll, b, h)
    return jnp.transpose(out_t, (2, 0, 1))


# speedup vs baseline: 15.4601x; 1.2115x over previous
import jax
import jax.numpy as jnp
from jax.experimental import pallas as pl
from jax.experimental.pallas import tpu as pltpu
from jax.experimental.pallas import tpu_sc as plsc

_WINDOW = 512
_PCHUNK = 2048
_OCHUNK = 512


def _dotT(x, eye):
    return jax.lax.dot_general(
        x,
        eye,
        (((0,), (0,)), ((), ())),
        precision=jax.lax.Precision.HIGHEST,
        preferred_element_type=jnp.float32,
    )


def _pack_body(xa_ref, xb_ref, o_ref):
    o_ref[...] = jnp.concatenate([xa_ref[...].T, xb_ref[...].T], axis=1)


def _pack_table(wt, npairs):
    h, v = wt.shape
    maxb = -(-v // _PCHUNK) - 1
    return pl.pallas_call(
        _pack_body,
        grid=(npairs,),
        in_specs=[
            pl.BlockSpec((h, _PCHUNK), lambda i: (0, 2 * i)),
            pl.BlockSpec(
                (h, _PCHUNK), lambda i: (0, jnp.minimum(2 * i + 1, maxb))
            ),
        ],
        out_specs=pl.BlockSpec((_PCHUNK, 2 * h), lambda i: (i, 0)),
        out_shape=jax.ShapeDtypeStruct((npairs * _PCHUNK, 2 * h), wt.dtype),
    )(wt, wt)


def _make_untranspose_body(h, half_b):
    def body(x_ref, o_ref):
        x = x_ref[...]
        o_ref[0, :, :half_b] = x[:, :h].T
        o_ref[0, :, half_b:] = x[:, h:].T

    return body


def _rows_to_out(rows2, ll, b, h):
    hb = b // 2
    return pl.pallas_call(
        _make_untranspose_body(h, hb),
        grid=(ll,),
        in_specs=[pl.BlockSpec((hb, 2 * h), lambda l: (l, 0))],
        out_specs=pl.BlockSpec((1, h, b), lambda l: (l, 0, 0)),
        out_shape=jax.ShapeDtypeStruct((ll, h, b), rows2.dtype),
    )(rows2)


def _sc_gather(table, idx, n, h):
    mesh = plsc.VectorSubcoreMesh(
        core_axis_name="core", subcore_axis_name="subcore"
    )

    @pl.kernel(
        out_type=jax.ShapeDtypeStruct((n, h), table.dtype),
        mesh=mesh,
        compiler_params=pltpu.CompilerParams(use_tc_tiling_on_sc=False),
    )
    def run(table_hbm, idx_hbm, out_hbm):
        def body(i_vmem, o_vmem):
            pltpu.sync_copy(table_hbm.at[i_vmem.at[0]], o_vmem)

        pltpu.emit_pipeline(
            body,
            grid=(n // _WINDOW,),
            in_specs=[
                pl.BlockSpec((1, _WINDOW), index_map=lambda i: (0, i))
            ],
            out_specs=[
                pl.BlockSpec((_WINDOW, h), index_map=lambda i: (i, 0))
            ],
            core_axis_name=("core", "subcore"),
            dimension_semantics=(pltpu.PARALLEL,),
        )(idx_hbm, out_hbm)

    return run(table, idx)


def kernel(inputs, weight):
    b, ll = inputs.shape
    v, h = weight.shape
    n = b * ll
    npairs = -(-v // (2 * _PCHUNK))
    vpad = npairs * 2 * _PCHUNK

    wt = weight.T

    packed = _pack_table(wt, npairs)
    table_lin = packed.reshape(vpad, h)

    s = jnp.arange(b, dtype=jnp.int32)
    perm = (s % 2) * (b // 2) + s // 2
    idx_t = inputs.T.astype(jnp.int32)
    be = idx_t // _PCHUNK
    j_t = ((be // 2) * _PCHUNK + (idx_t % _PCHUNK)) * 2 + (be % 2)
    j = jnp.take(j_t, perm, axis=1).reshape(1, n)

    rows = _sc_gather(table_lin, j, n, h)
    out_t = _rows_to_out(rows.reshape(n // 2, 2 * h), ll, b, h)
    return jnp.transpose(out_t, (2, 0, 1))
